# Initial kernel scaffold; baseline (speedup 1.0000x reference)
import jax
import jax.numpy as jnp
from jax.experimental import pallas as pl


def kernel(input_indices, position_indices, word_table, pos_table):
    raise NotImplementedError("write your pallas kernel here")


# speedup vs baseline: 1.3439x; 1.3439x over previous
import functools

import jax
import jax.numpy as jnp
from jax import lax
from jax.experimental import pallas as pl
from jax.experimental.pallas import tpu as pltpu
from jax.experimental.pallas import tpu_sc as plsc

HIDDEN = 768
B_TOTAL = 8192
NW = 32
B_PER_W = B_TOTAL // NW
CHUNK = 64
NCHUNK = B_PER_W // CHUNK
LANES = 16
COLS = HIDDEN // LANES


def _emb_body(widx_hbm, pidx_hbm, word_hbm, pos_hbm, out_hbm,
              idx_w, idx_p, buf_w, buf_p, sem_w, sem_p):
    wid = lax.axis_index("s") * 2 + lax.axis_index("c")
    base = wid * B_PER_W
    pltpu.sync_copy(widx_hbm.at[pl.ds(base, B_PER_W)], idx_w)
    pltpu.sync_copy(pidx_hbm.at[pl.ds(base, B_PER_W)], idx_p)
    for c in range(NCHUNK):
        cw = pltpu.async_copy(
            word_hbm.at[idx_w.at[pl.ds(c * CHUNK, CHUNK)]], buf_w, sem_w)
        cp = pltpu.async_copy(
            pos_hbm.at[idx_p.at[pl.ds(c * CHUNK, CHUNK)]], buf_p, sem_p)
        cw.wait()
        cp.wait()

        def row_body(r, carry):
            for j in range(COLS):
                sl = (r, pl.ds(j * LANES, LANES))
                buf_w[sl] = buf_w[sl] + buf_p[sl]
            return carry

        lax.fori_loop(0, CHUNK, row_body, 0)
        pltpu.sync_copy(buf_w, out_hbm.at[pl.ds(base + c * CHUNK, CHUNK)])


@jax.jit
def _run(widx, pidx, word_table, pos_table):
    mesh = plsc.VectorSubcoreMesh(core_axis_name="c", subcore_axis_name="s")
    k = functools.partial(
        pl.kernel,
        mesh=mesh,
        out_type=jax.ShapeDtypeStruct((B_TOTAL, HIDDEN), jnp.float32),
        scratch_types=[
            pltpu.VMEM((B_PER_W,), jnp.int32),
            pltpu.VMEM((B_PER_W,), jnp.int32),
            pltpu.VMEM((CHUNK, HIDDEN), jnp.float32),
            pltpu.VMEM((CHUNK, HIDDEN), jnp.float32),
            pltpu.SemaphoreType.DMA,
            pltpu.SemaphoreType.DMA,
        ],
    )(_emb_body)
    return k(widx, pidx, word_table, pos_table)


def kernel(input_indices, position_indices, word_table, pos_table):
    widx = input_indices.reshape(-1).astype(jnp.int32)
    pidx = position_indices.reshape(-1).astype(jnp.int32)
    out = _run(widx, pidx, word_table, pos_table)
    return out.reshape(input_indices.shape + (HIDDEN,))


# speedup vs baseline: 1.5817x; 1.1769x over previous
import functools

import jax
import jax.numpy as jnp
from jax import lax
from jax.experimental import pallas as pl
from jax.experimental.pallas import tpu as pltpu
from jax.experimental.pallas import tpu_sc as plsc

HIDDEN = 768
B_TOTAL = 8192
NW = 32
B_PER_W = B_TOTAL // NW
CHUNK = 32
NCHUNK = B_PER_W // CHUNK
LANES = 16
COLS = HIDDEN // LANES


def _emb_body(widx_hbm, pidx_hbm, word_hbm, pos_hbm, out_hbm,
              idx_w, idx_p, bw, bp,
              sem_w0, sem_w1, sem_p0, sem_p1, sem_o0, sem_o1):
    wid = lax.axis_index("s") * 2 + lax.axis_index("c")
    base = wid * B_PER_W
    pltpu.sync_copy(widx_hbm.at[pl.ds(base, B_PER_W)], idx_w)
    pltpu.sync_copy(pidx_hbm.at[pl.ds(base, B_PER_W)], idx_p)

    sems_w = (sem_w0, sem_w1)
    sems_p = (sem_p0, sem_p1)
    sems_o = (sem_o0, sem_o1)
    gath = [None, None]
    outd = [None, None]

    for c in range(NCHUNK + 1):
        k = c % 2
        if c < NCHUNK:
            if outd[k] is not None:
                outd[k].wait()
            gath[k] = (
                pltpu.async_copy(
                    word_hbm.at[idx_w.at[pl.ds(c * CHUNK, CHUNK)]],
                    bw.at[k], sems_w[k]),
                pltpu.async_copy(
                    pos_hbm.at[idx_p.at[pl.ds(c * CHUNK, CHUNK)]],
                    bp.at[k], sems_p[k]),
            )
        if c >= 1:
            kp = (c - 1) % 2
            gath[kp][0].wait()
            gath[kp][1].wait()

            def row_body(r, carry, kp=kp):
                for j in range(COLS):
                    sl = (r, pl.ds(j * LANES, LANES))
                    plsc.addupdate(bw.at[kp].at[sl], bp.at[kp][sl])
                return carry

            lax.fori_loop(0, CHUNK, row_body, 0)
            outd[kp] = pltpu.async_copy(
                bw.at[kp],
                out_hbm.at[pl.ds(base + (c - 1) * CHUNK, CHUNK)],
                sems_o[kp])
    for k in range(2):
        if outd[k] is not None:
            outd[k].wait()


@jax.jit
def _run(widx, pidx, word_table, pos_table):
    mesh = plsc.VectorSubcoreMesh(core_axis_name="c", subcore_axis_name="s")
    k = functools.partial(
        pl.kernel,
        mesh=mesh,
        out_type=jax.ShapeDtypeStruct((B_TOTAL, HIDDEN), jnp.float32),
        scratch_types=[
            pltpu.VMEM((B_PER_W,), jnp.int32),
            pltpu.VMEM((B_PER_W,), jnp.int32),
            pltpu.VMEM((2, CHUNK, HIDDEN), jnp.float32),
            pltpu.VMEM((2, CHUNK, HIDDEN), jnp.float32),
            pltpu.SemaphoreType.DMA,
            pltpu.SemaphoreType.DMA,
            pltpu.SemaphoreType.DMA,
            pltpu.SemaphoreType.DMA,
            pltpu.SemaphoreType.DMA,
            pltpu.SemaphoreType.DMA,
        ],
    )(_emb_body)
    return k(widx, pidx, word_table, pos_table)


def kernel(input_indices, position_indices, word_table, pos_table):
    widx = input_indices.reshape(-1).astype(jnp.int32)
    pidx = position_indices.reshape(-1).astype(jnp.int32)
    out = _run(widx, pidx, word_table, pos_table)
    return out.reshape(input_indices.shape + (HIDDEN,))
